# baseline (speedup 1.0000x reference)
import jax
import jax.numpy as jnp
from jax.experimental import pallas as pl
from jax.experimental.pallas import tpu as pltpu

_N = 512


def _cnn_kernel(x_ref, w1b_ref, b1_ref, w2b_ref, b2_ref, wfc_ref, bfc_ref,
                out_ref, xpad_ref, h1_ref, s1_ref, o1_ref, s2_ref, o2_ref,
                f_ref):

    xpad_ref[...] = jnp.zeros_like(xpad_ref)
    h1_ref[...] = jnp.zeros_like(h1_ref)
    s1_ref[:, pl.ds(540, 4), :] = jnp.zeros((2, 4, _N), jnp.bfloat16)

    for i in range(28):
        xpad_ref[pl.ds(30 * (i + 1) + 1, 28), :] = x_ref[pl.ds(28 * i, 28), :]

    for a in range(14):
        u = a % 2
        base = 60 * a + 31
        for t in range(9):
            dy, dx = divmod(t, 3)
            s1_ref[u, pl.ds(60 * t, 60), :] = (
                xpad_ref[pl.ds(base + 30 * (dy - 1) + (dx - 1), 60), :]
                .astype(jnp.bfloat16))
        o1_ref[u, ...] = jnp.dot(w1b_ref[...], s1_ref[u, ...],
                                 preferred_element_type=jnp.float32)
        m = jnp.maximum(
            jnp.maximum(o1_ref[u, pl.ds(0, 112), :],
                        o1_ref[u, pl.ds(112, 112), :]),
            jnp.maximum(o1_ref[u, pl.ds(224, 112), :],
                        o1_ref[u, pl.ds(336, 112), :]))
        for c in range(8):
            h1_ref[pl.ds(c * 264 + 16 * (a + 1) + 1, 14), :] = (
                jnp.maximum(m[c * 14:c * 14 + 14] + b1_ref[0, c], 0.0))

    for a in range(7):
        u = a % 2
        for ci in range(8):
            for t in range(9):
                dy, dx = divmod(t, 3)
                src = ci * 264 + 17 + 16 * (dy - 1) + (dx - 1) + 32 * a
                s2_ref[u, pl.ds(32 * (ci * 9 + t), 32), :] = (
                    h1_ref[pl.ds(src, 32), :].astype(jnp.bfloat16))
        o2_ref[u, ...] = jnp.dot(w2b_ref[...], s2_ref[u, ...],
                                 preferred_element_type=jnp.float32)
        m = jnp.maximum(
            jnp.maximum(o2_ref[u, pl.ds(0, 112), :],
                        o2_ref[u, pl.ds(112, 112), :]),
            jnp.maximum(o2_ref[u, pl.ds(224, 112), :],
                        o2_ref[u, pl.ds(336, 112), :]))
        for co in range(16):
            f_ref[pl.ds(co * 49 + 7 * a, 7), :] = (
                jnp.maximum(m[co * 7:co * 7 + 7] + b2_ref[0, co],
                            0.0).astype(jnp.bfloat16))

    out_ref[...] = jnp.dot(wfc_ref[...], f_ref[...],
                           preferred_element_type=jnp.float32) + bfc_ref[...]


def _forward(x, w1p, b1p, w2p, b2p, wfc_p, bfc_p):
    B = x.shape[0]
    G = B // _N
    bf16 = jnp.bfloat16

    xT = jnp.transpose(x.reshape(B, 784))
    b1s = b1p[:, :8]
    b2s = b2p[:, :16]
    q = jnp.arange(4)
    w1s = w1p[:, 0, :8]
    sel1 = (30 * (q // 2)[:, None, None] + 2 * jnp.arange(14)[None, :, None]
            + (q % 2)[:, None, None])
    e1 = (jnp.arange(60)[None, None, :] == sel1).astype(jnp.float32)
    w1b = jnp.einsum('tc,qbr->qcbtr', w1s, e1).reshape(448, 540)
    w1b = jnp.pad(w1b, ((0, 0), (0, 4))).astype(bf16)
    w2k = jnp.transpose(w2p[:, :, :16], (1, 0, 2)).reshape(72, 16)
    sel2 = (16 * (q // 2)[:, None, None] + 2 * jnp.arange(7)[None, :, None]
            + (q % 2)[:, None, None])
    e2 = (jnp.arange(32)[None, None, :] == sel2).astype(jnp.float32)
    w2b = jnp.einsum('kc,qbr->qcbkr', w2k, e2).reshape(448, 2304).astype(bf16)
    wfc_t = jnp.transpose(wfc_p.reshape(49, 16, 10), (2, 1, 0)).reshape(10, 784)
    wfc16 = jnp.pad(wfc_t, ((0, 6), (0, 0))).astype(bf16)
    bfc16 = jnp.pad(bfc_p, ((0, 0), (0, 6)))
    bfcv = jnp.broadcast_to(bfc16.reshape(16, 1), (16, _N))

    out = pl.pallas_call(
        _cnn_kernel,
        out_shape=jax.ShapeDtypeStruct((G, 16, _N), jnp.float32),
        grid=(G,),
        in_specs=[
            pl.BlockSpec((784, _N), lambda g: (0, g)),
            pl.BlockSpec((448, 544), lambda g: (0, 0)),
            pl.BlockSpec(memory_space=pltpu.SMEM),
            pl.BlockSpec((448, 2304), lambda g: (0, 0)),
            pl.BlockSpec(memory_space=pltpu.SMEM),
            pl.BlockSpec((16, 784), lambda g: (0, 0)),
            pl.BlockSpec((16, _N), lambda g: (0, 0)),
        ],
        out_specs=pl.BlockSpec((None, 16, _N), lambda g: (g, 0, 0)),
        scratch_shapes=[
            pltpu.VMEM((904, _N), jnp.float32),
            pltpu.VMEM((2112, _N), jnp.float32),
            pltpu.VMEM((2, 544, _N), bf16),
            pltpu.VMEM((2, 448, _N), jnp.float32),
            pltpu.VMEM((2, 2304, _N), bf16),
            pltpu.VMEM((2, 448, _N), jnp.float32),
            pltpu.VMEM((784, _N), bf16),
        ],
        compiler_params=pltpu.CompilerParams(
            dimension_semantics=("arbitrary",)),
    )(xT, w1b, b1s, w2b, b2s, wfc16, bfcv)

    return jnp.transpose(out, (0, 2, 1)).reshape(B, 16)[:, :10]


_forward_jit = jax.jit(_forward)


def kernel(x, w1p, b1p, w2p, b2p, wfc_p, bfc_p):
    return _forward_jit(x, w1p, b1p, w2p, b2p, wfc_p, bfc_p)


# speedup vs baseline: 1.7430x; 1.7430x over previous
import jax
import jax.numpy as jnp
from jax.experimental import pallas as pl
from jax.experimental.pallas import tpu as pltpu

_N = 256


def _cnn_kernel(x_ref, w1b_ref, b1_ref, w2b_ref, b2_ref, wfc_ref, bfc_ref,
                out_ref, xpad_ref, h1_ref, s1_ref, o1_ref, s2_ref, o2_ref,
                f_ref):

    xpad_ref[...] = jnp.zeros_like(xpad_ref)
    h1_ref[...] = jnp.zeros_like(h1_ref)
    s1_ref[:, pl.ds(180, 12), :] = jnp.zeros((2, 12, _N), jnp.bfloat16)

    for i in range(28):
        xpad_ref[pl.ds(30 * (i + 1) + 1, 28), :] = x_ref[pl.ds(28 * i, 28), :]

    for a in range(14):
        u = a % 2
        for ky in range(3):
            s1_ref[u, pl.ds(60 * ky, 60), :] = (
                xpad_ref[pl.ds(60 * a + 30 * ky, 60), :]
                .astype(jnp.bfloat16))
        o1_ref[u, ...] = jnp.dot(w1b_ref[...], s1_ref[u, ...],
                                 preferred_element_type=jnp.float32)
        m = jnp.maximum(
            jnp.maximum(o1_ref[u, pl.ds(0, 112), :],
                        o1_ref[u, pl.ds(112, 112), :]),
            jnp.maximum(o1_ref[u, pl.ds(224, 112), :],
                        o1_ref[u, pl.ds(336, 112), :]))
        for c in range(8):
            h1_ref[pl.ds(c * 264 + 16 * (a + 1) + 1, 14), :] = (
                jnp.maximum(m[c * 14:c * 14 + 14] + b1_ref[0, c], 0.0))

    for a in range(7):
        u = a % 2
        for ci in range(8):
            for ky in range(3):
                s2_ref[u, pl.ds(32 * (ci * 3 + ky), 32), :] = (
                    h1_ref[pl.ds(ci * 264 + 32 * a + 16 * ky, 32), :]
                    .astype(jnp.bfloat16))
        o2_ref[u, ...] = jnp.dot(w2b_ref[...], s2_ref[u, ...],
                                 preferred_element_type=jnp.float32)
        m = jnp.maximum(
            jnp.maximum(o2_ref[u, pl.ds(0, 112), :],
                        o2_ref[u, pl.ds(112, 112), :]),
            jnp.maximum(o2_ref[u, pl.ds(224, 112), :],
                        o2_ref[u, pl.ds(336, 112), :]))
        for co in range(16):
            f_ref[pl.ds(co * 49 + 7 * a, 7), :] = (
                jnp.maximum(m[co * 7:co * 7 + 7] + b2_ref[0, co],
                            0.0).astype(jnp.bfloat16))

    out_ref[...] = jnp.dot(wfc_ref[...], f_ref[...],
                           preferred_element_type=jnp.float32) + bfc_ref[...]


def _forward(x, w1p, b1p, w2p, b2p, wfc_p, bfc_p):
    B = x.shape[0]
    G = B // _N
    bf16 = jnp.bfloat16

    xT = jnp.transpose(x.reshape(B, 784))
    b1s = b1p[:, :8]
    b2s = b2p[:, :16]
    q = jnp.arange(4)
    kx = jnp.arange(3)[:, None, None, None]
    w1s = w1p[:, 0, :8].reshape(3, 3, 8)
    sel1 = (30 * (q // 2)[:, None, None] + 2 * jnp.arange(14)[None, :, None]
            + (q % 2)[:, None, None])
    e1 = (jnp.arange(60)[None, None, None, :] == sel1[None] + kx
          ).astype(jnp.float32)
    w1b = jnp.einsum('ykc,kqbr->qcbyr', w1s, e1).reshape(448, 180)
    w1b = jnp.pad(w1b, ((0, 0), (0, 12))).astype(bf16)
    w2k = jnp.transpose(w2p[:, :, :16], (1, 0, 2)).reshape(8, 3, 3, 16)
    sel2 = (16 * (q // 2)[:, None, None] + 2 * jnp.arange(7)[None, :, None]
            + (q % 2)[:, None, None])
    e2 = (jnp.arange(32)[None, None, None, :] == sel2[None] + kx
          ).astype(jnp.float32)
    w2b = jnp.einsum('iyko,kqbr->qobiyr', w2k, e2).reshape(448, 768)
    w2b = w2b.astype(bf16)
    wfc_t = jnp.transpose(wfc_p.reshape(49, 16, 10), (2, 1, 0)).reshape(10, 784)
    wfc16 = jnp.pad(wfc_t, ((0, 6), (0, 0))).astype(bf16)
    bfc16 = jnp.pad(bfc_p, ((0, 0), (0, 6)))
    bfcv = jnp.broadcast_to(bfc16.reshape(16, 1), (16, _N))

    out = pl.pallas_call(
        _cnn_kernel,
        out_shape=jax.ShapeDtypeStruct((G, 16, _N), jnp.float32),
        grid=(G,),
        in_specs=[
            pl.BlockSpec((784, _N), lambda g: (0, g)),
            pl.BlockSpec((448, 192), lambda g: (0, 0)),
            pl.BlockSpec(memory_space=pltpu.SMEM),
            pl.BlockSpec((448, 768), lambda g: (0, 0)),
            pl.BlockSpec(memory_space=pltpu.SMEM),
            pl.BlockSpec((16, 784), lambda g: (0, 0)),
            pl.BlockSpec((16, _N), lambda g: (0, 0)),
        ],
        out_specs=pl.BlockSpec((None, 16, _N), lambda g: (g, 0, 0)),
        scratch_shapes=[
            pltpu.VMEM((904, _N), jnp.float32),
            pltpu.VMEM((2112, _N), jnp.float32),
            pltpu.VMEM((2, 192, _N), bf16),
            pltpu.VMEM((2, 448, _N), jnp.float32),
            pltpu.VMEM((2, 768, _N), bf16),
            pltpu.VMEM((2, 448, _N), jnp.float32),
            pltpu.VMEM((784, _N), bf16),
        ],
        compiler_params=pltpu.CompilerParams(
            dimension_semantics=("arbitrary",)),
    )(xT, w1b, b1s, w2b, b2s, wfc16, bfcv)

    return jnp.transpose(out, (0, 2, 1)).reshape(B, 16)[:, :10]


_forward_jit = jax.jit(_forward)


def kernel(x, w1p, b1p, w2p, b2p, wfc_p, bfc_p):
    return _forward_jit(x, w1p, b1p, w2p, b2p, wfc_p, bfc_p)
